# baseline (speedup 1.0000x reference)
import functools
import jax
import jax.numpy as jnp
from jax import lax
from jax.experimental import pallas as pl
from jax.experimental.pallas import tpu as pltpu, tpu_sc as plsc

_CHUNK = 16


def _make_sc(B, S, D):
    info = plsc.get_sparse_core_info()
    NC, NS, L = info.num_cores, info.num_subcores, info.num_lanes
    NW = NC * NS
    s_per_w = S // NW
    n_chunks = s_per_w // _CHUNK
    vregs_per_row = D // L
    mesh = plsc.VectorSubcoreMesh(core_axis_name="c", subcore_axis_name="s")

    @functools.partial(
        pl.kernel,
        mesh=mesh,
        out_type=jax.ShapeDtypeStruct((B, S, D), jnp.float32),
        scratch_types=[
            pltpu.VMEM((_CHUNK, D), jnp.float32),
            pltpu.VMEM((_CHUNK, D), jnp.float32),
        ],
    )
    def k(x_hbm, pos_hbm, out_hbm, pos_v, x_v):
        wid = lax.axis_index("s") * NC + lax.axis_index("c")
        base = wid * s_per_w

        def chunk_body(t, carry):
            s0 = base + t * _CHUNK
            pltpu.sync_copy(pos_hbm.at[pl.ds(s0, _CHUNK)], pos_v)

            def batch_body(b, carry2):
                pltpu.sync_copy(x_hbm.at[b, pl.ds(s0, _CHUNK)], x_v)

                def add_body(i, carry3):
                    r = i // vregs_per_row
                    c = (i % vregs_per_row) * L
                    x_v[r, pl.ds(c, L)] = x_v[r, pl.ds(c, L)] + pos_v[r, pl.ds(c, L)]
                    return carry3

                lax.fori_loop(0, _CHUNK * vregs_per_row, add_body, 0, unroll=4)
                pltpu.sync_copy(x_v, out_hbm.at[b, pl.ds(s0, _CHUNK)])
                return carry2

            lax.fori_loop(0, B, batch_body, 0)
            return carry

        lax.fori_loop(0, n_chunks, chunk_body, 0)

    return k


def kernel(x, pos_table):
    B, S, D = x.shape
    pos = pos_table[:S]
    return _make_sc(B, S, D)(x, pos)
